# Initial kernel scaffold; baseline (speedup 1.0000x reference)
import functools

import jax
import jax.numpy as jnp
from jax import lax
from jax.experimental import pallas as pl
from jax.experimental.pallas import tpu as pltpu
from jax.experimental.pallas import tpu_sc as plsc

_NC = 2
_NS = 16
_NW = _NC * _NS
_RPD = 128


def _sc_gather(x3, table):
    n_rows = x3.shape[0]
    hid = table.shape[1]
    rows_per_w = n_rows // _NW
    R = 8
    steps = rows_per_w // R

    mesh = plsc.VectorSubcoreMesh(
        core_axis_name="c", subcore_axis_name="s",
        num_cores=_NC, num_subcores=_NS)

    @functools.partial(
        pl.kernel,
        out_type=jax.ShapeDtypeStruct((n_rows, _RPD, hid), jnp.float32),
        mesh=mesh,
        scratch_types=[
            pltpu.VMEM((R, _RPD), jnp.int32),
            pltpu.VMEM((R, _RPD, hid), jnp.float32),
            pltpu.SemaphoreType.DMA,
        ],
    )
    def gather_k(x_hbm, table_hbm, out_hbm, idx_v, rows_v, sem):
        wid = lax.axis_index("s") * _NC + lax.axis_index("c")
        base = wid * rows_per_w

        def body(i, carry):
            row0 = base + i * R
            pltpu.sync_copy(x_hbm.at[pl.ds(row0, R)], idx_v)
            copies = [
                pltpu.async_copy(table_hbm.at[idx_v.at[j]], rows_v.at[j], sem)
                for j in range(R)
            ]
            for c in copies:
                c.wait()
            pltpu.sync_copy(rows_v, out_hbm.at[pl.ds(row0, R)])
            return carry

        lax.fori_loop(0, steps, body, 0)

    return gather_k(x3, table)


def _tc_project(h, wt, b2):
    n, hid = h.shape
    emb = wt.shape[1]
    blk = 4096
    grid = n // blk

    def proj_k(h_ref, wt_ref, b_ref, o_ref):
        o_ref[...] = (
            jnp.dot(h_ref[...], wt_ref[...],
                    preferred_element_type=jnp.float32)
            + b_ref[...]
        )

    return pl.pallas_call(
        proj_k,
        grid=(grid,),
        in_specs=[
            pl.BlockSpec((blk, hid), lambda i: (i, 0)),
            pl.BlockSpec((hid, emb), lambda i: (0, 0)),
            pl.BlockSpec((1, emb), lambda i: (0, 0)),
        ],
        out_specs=pl.BlockSpec((blk, emb), lambda i: (i, 0)),
        out_shape=jax.ShapeDtypeStruct((n, emb), jnp.float32),
    )(h, wt, b2)


def kernel(x, table, W, b):
    bsz, seq = x.shape
    n = bsz * seq
    emb, hid = W.shape
    x3 = x.astype(jnp.int32).reshape(n // _RPD, _RPD)
    h = _sc_gather(x3, table).reshape(n, hid)
    out = _tc_project(h, W.T, b.reshape(1, emb))
    return out.reshape(bsz, seq, emb)


# speedup vs baseline: 17.8373x; 17.8373x over previous
import functools

import jax
import jax.numpy as jnp
from jax import lax
from jax.experimental import pallas as pl
from jax.experimental.pallas import tpu as pltpu
from jax.experimental.pallas import tpu_sc as plsc

_NC = 2
_NS = 16
_NW = _NC * _NS
_RPD = 128


def _sc_gather(x3, table):
    n_rows = x3.shape[0]
    hid = table.shape[1]
    rows_per_w = n_rows // _NW
    R = 8
    steps = rows_per_w // R

    mesh = plsc.VectorSubcoreMesh(
        core_axis_name="c", subcore_axis_name="s",
        num_cores=_NC, num_subcores=_NS)

    @functools.partial(
        pl.kernel,
        out_type=jax.ShapeDtypeStruct((n_rows, _RPD, hid), jnp.float32),
        mesh=mesh,
        scratch_types=[
            pltpu.VMEM((R, _RPD), jnp.int32),
            pltpu.VMEM((R, _RPD, hid), jnp.float32),
            pltpu.SemaphoreType.DMA,
        ],
        compiler_params=pltpu.CompilerParams(use_tc_tiling_on_sc=False),
    )
    def gather_k(x_hbm, table_hbm, out_hbm, idx_v, rows_v, sem):
        wid = lax.axis_index("s") * _NC + lax.axis_index("c")
        base = wid * rows_per_w

        def body(i, carry):
            row0 = base + i * R
            pltpu.sync_copy(x_hbm.at[pl.ds(row0, R)], idx_v)
            copies = [
                pltpu.async_copy(table_hbm.at[idx_v.at[j]], rows_v.at[j], sem)
                for j in range(R)
            ]
            for c in copies:
                c.wait()
            pltpu.sync_copy(rows_v, out_hbm.at[pl.ds(row0, R)])
            return carry

        lax.fori_loop(0, steps, body, 0)

    return gather_k(x3, table)


def _tc_project(h, wt, b2):
    n, hid = h.shape
    emb = wt.shape[1]
    blk = 4096
    grid = n // blk

    def proj_k(h_ref, wt_ref, b_ref, o_ref):
        o_ref[...] = (
            jnp.dot(h_ref[...], wt_ref[...],
                    preferred_element_type=jnp.float32)
            + b_ref[...]
        )

    return pl.pallas_call(
        proj_k,
        grid=(grid,),
        in_specs=[
            pl.BlockSpec((blk, hid), lambda i: (i, 0)),
            pl.BlockSpec((hid, emb), lambda i: (0, 0)),
            pl.BlockSpec((1, emb), lambda i: (0, 0)),
        ],
        out_specs=pl.BlockSpec((blk, emb), lambda i: (i, 0)),
        out_shape=jax.ShapeDtypeStruct((n, emb), jnp.float32),
    )(h, wt, b2)


def kernel(x, table, W, b):
    bsz, seq = x.shape
    n = bsz * seq
    emb, hid = W.shape
    x3 = x.astype(jnp.int32).reshape(n // _RPD, _RPD)
    h = _sc_gather(x3, table).reshape(n, hid)
    out = _tc_project(h, W.T, b.reshape(1, emb))
    return out.reshape(bsz, seq, emb)


# speedup vs baseline: 25.7094x; 1.4413x over previous
import functools

import jax
import jax.numpy as jnp
from jax import lax
from jax.experimental import pallas as pl
from jax.experimental.pallas import tpu as pltpu
from jax.experimental.pallas import tpu_sc as plsc

_NC = 2
_NS = 16
_NW = _NC * _NS
_RPD = 128


def _tc_build_p(tableT, wt, b2, vp, vb):
    hid, _ = tableT.shape
    emb = wt.shape[1]
    grid = vp // vb

    def pk(t_ref, w_ref, b_ref, o_ref):
        o_ref[...] = (
            lax.dot_general(t_ref[...], w_ref[...],
                            (((0,), (0,)), ((), ())),
                            preferred_element_type=jnp.float32)
            + b_ref[...]
        )

    return pl.pallas_call(
        pk,
        grid=(grid,),
        in_specs=[
            pl.BlockSpec((hid, vb), lambda i: (0, i)),
            pl.BlockSpec((hid, emb), lambda i: (0, 0)),
            pl.BlockSpec((1, emb), lambda i: (0, 0)),
        ],
        out_specs=pl.BlockSpec((vb, emb), lambda i: (i, 0)),
        out_shape=jax.ShapeDtypeStruct((vp, emb), jnp.float32),
    )(tableT, wt, b2)


def _sc_gather(x3, p):
    n_rows = x3.shape[0]
    emb = p.shape[1]
    rows_per_w = n_rows // _NW
    R = 4
    steps = rows_per_w // R

    mesh = plsc.VectorSubcoreMesh(
        core_axis_name="c", subcore_axis_name="s",
        num_cores=_NC, num_subcores=_NS)

    @functools.partial(
        pl.kernel,
        out_type=jax.ShapeDtypeStruct((n_rows, _RPD, emb), jnp.float32),
        mesh=mesh,
        scratch_types=[
            pltpu.VMEM((R, _RPD), jnp.int32),
            pltpu.VMEM((R, _RPD, emb), jnp.float32),
            pltpu.SemaphoreType.DMA,
        ],
    )
    def gather_k(x_hbm, p_hbm, out_hbm, idx_v, rows_v, sem):
        wid = lax.axis_index("s") * _NC + lax.axis_index("c")
        base = wid * rows_per_w

        def body(i, carry):
            row0 = base + i * R
            pltpu.sync_copy(x_hbm.at[pl.ds(row0, R)], idx_v)
            copies = [
                pltpu.async_copy(p_hbm.at[idx_v.at[j]], rows_v.at[j], sem)
                for j in range(R)
            ]
            for c in copies:
                c.wait()
            pltpu.sync_copy(rows_v, out_hbm.at[pl.ds(row0, R)])
            return carry

        lax.fori_loop(0, steps, body, 0)

    return gather_k(x3, p)


def kernel(x, table, W, b):
    bsz, seq = x.shape
    n = bsz * seq
    emb, hid = W.shape
    vocab = table.shape[0]
    vb = 2048
    vp = ((vocab + vb - 1) // vb) * vb
    p = _tc_build_p(table.T, W.T, b.reshape(1, emb), vp, vb)
    x3 = x.astype(jnp.int32).reshape(n // _RPD, _RPD)
    out = _sc_gather(x3, p)
    return out.reshape(bsz, seq, emb)


# speedup vs baseline: 26.9844x; 1.0496x over previous
import functools

import jax
import jax.numpy as jnp
from jax import lax
from jax.experimental import pallas as pl
from jax.experimental.pallas import tpu as pltpu
from jax.experimental.pallas import tpu_sc as plsc

_NC = 2
_NS = 16
_NW = _NC * _NS
_RPD = 128


def _tc_build_p(tableT, wt, b2, vp, vb):
    hid, _ = tableT.shape
    emb = wt.shape[1]
    grid = vp // vb

    def pk(t_ref, w_ref, b_ref, o_ref):
        o_ref[...] = (
            lax.dot_general(t_ref[...], w_ref[...],
                            (((0,), (0,)), ((), ())),
                            preferred_element_type=jnp.float32)
            + b_ref[...]
        )

    return pl.pallas_call(
        pk,
        grid=(grid,),
        in_specs=[
            pl.BlockSpec((hid, vb), lambda i: (0, i)),
            pl.BlockSpec((hid, emb), lambda i: (0, 0)),
            pl.BlockSpec((1, emb), lambda i: (0, 0)),
        ],
        out_specs=pl.BlockSpec((vb, emb), lambda i: (i, 0)),
        out_shape=jax.ShapeDtypeStruct((vp, emb), jnp.float32),
    )(tableT, wt, b2)


def _sc_gather(x3, p):
    n_rows = x3.shape[0]
    emb = p.shape[1]
    rows_per_w = n_rows // _NW
    R = 2
    steps = rows_per_w // R
    half = steps // 2

    mesh = plsc.VectorSubcoreMesh(
        core_axis_name="c", subcore_axis_name="s",
        num_cores=_NC, num_subcores=_NS)

    @functools.partial(
        pl.kernel,
        out_type=jax.ShapeDtypeStruct((n_rows, _RPD, emb), jnp.float32),
        mesh=mesh,
        scratch_types=[
            pltpu.VMEM((2, R, _RPD), jnp.int32),
            pltpu.VMEM((2, R, _RPD, emb), jnp.float32),
            pltpu.SemaphoreType.DMA,
            pltpu.SemaphoreType.DMA,
            pltpu.SemaphoreType.DMA,
            pltpu.SemaphoreType.DMA,
        ],
    )
    def gather_k(x_hbm, p_hbm, out_hbm, idx_v, rows_v, sg0, sg1, sw0, sw1):
        wid = lax.axis_index("s") * _NC + lax.axis_index("c")
        base = wid * rows_per_w
        sg = (sg0, sg1)
        sw = (sw0, sw1)

        def fire(step, b):
            row0 = base + step * R
            pltpu.sync_copy(x_hbm.at[pl.ds(row0, R)], idx_v.at[b])
            for j in range(R):
                pltpu.make_async_copy(
                    p_hbm.at[idx_v.at[b, j]], rows_v.at[b, j], sg[b]
                ).start()

        def drain_gather(b):
            for j in range(R):
                pltpu.make_async_copy(
                    p_hbm.at[idx_v.at[b, j]], rows_v.at[b, j], sg[b]
                ).wait()

        def wb_start(step, b):
            row0 = base + step * R
            pltpu.make_async_copy(
                rows_v.at[b], out_hbm.at[pl.ds(row0, R)], sw[b]
            ).start()

        def wb_wait(step, b):
            row0 = base + step * R
            pltpu.make_async_copy(
                rows_v.at[b], out_hbm.at[pl.ds(row0, R)], sw[b]
            ).wait()

        fire(0, 0)
        fire(1, 1)

        def body(i, carry):
            g = 2 * i
            drain_gather(0)
            wb_start(g, 0)
            wb_wait(g, 0)
            @pl.when(i < half - 1)
            def _():
                fire(g + 2, 0)
            drain_gather(1)
            wb_start(g + 1, 1)
            wb_wait(g + 1, 1)
            @pl.when(i < half - 1)
            def _():
                fire(g + 3, 1)
            return carry

        lax.fori_loop(0, half, body, 0)

    return gather_k(x3, p)


def kernel(x, table, W, b):
    bsz, seq = x.shape
    n = bsz * seq
    emb, hid = W.shape
    vocab = table.shape[0]
    vb = 2048
    vp = ((vocab + vb - 1) // vb) * vb
    p = _tc_build_p(table.T, W.T, b.reshape(1, emb), vp, vb)
    x3 = x.astype(jnp.int32).reshape(n // _RPD, _RPD)
    out = _sc_gather(x3, p)
    return out.reshape(bsz, seq, emb)


# speedup vs baseline: 32.7950x; 1.2153x over previous
import functools

import jax
import jax.numpy as jnp
from jax import lax
from jax.experimental import pallas as pl
from jax.experimental.pallas import tpu as pltpu
from jax.experimental.pallas import tpu_sc as plsc

_NC = 2
_NS = 16
_NW = _NC * _NS
_RPD = 128


def _tc_build_p(tableT, wt, b2, vp, vb):
    hid, _ = tableT.shape
    emb = wt.shape[1]
    grid = vp // vb

    def pk(t_ref, w_ref, b_ref, o_ref):
        o_ref[...] = (
            lax.dot_general(t_ref[...], w_ref[...],
                            (((0,), (0,)), ((), ())),
                            preferred_element_type=jnp.float32)
            + b_ref[...]
        )

    return pl.pallas_call(
        pk,
        grid=(grid,),
        in_specs=[
            pl.BlockSpec((hid, vb), lambda i: (0, i)),
            pl.BlockSpec((hid, emb), lambda i: (0, 0)),
            pl.BlockSpec((1, emb), lambda i: (0, 0)),
        ],
        out_specs=pl.BlockSpec((vb, emb), lambda i: (i, 0)),
        out_shape=jax.ShapeDtypeStruct((vp, emb), jnp.float32),
    )(tableT, wt, b2)


def _sc_gather(x3, p):
    n_rows = x3.shape[0]
    emb = p.shape[1]
    rows_per_w = n_rows // _NW
    R = 2
    steps = rows_per_w // R
    half = steps // 2

    mesh = plsc.VectorSubcoreMesh(
        core_axis_name="c", subcore_axis_name="s",
        num_cores=_NC, num_subcores=_NS)

    @functools.partial(
        pl.kernel,
        out_type=jax.ShapeDtypeStruct((n_rows, _RPD, emb), jnp.float32),
        mesh=mesh,
        scratch_types=[
            pltpu.VMEM((2, R, _RPD), jnp.int32),
            pltpu.VMEM((2, R, _RPD, emb), jnp.float32),
            pltpu.SemaphoreType.DMA,
            pltpu.SemaphoreType.DMA,
            pltpu.SemaphoreType.DMA,
            pltpu.SemaphoreType.DMA,
        ],
    )
    def gather_k(x_hbm, p_hbm, out_hbm, idx_v, rows_v, sg0, sg1, sw0, sw1):
        wid = lax.axis_index("s") * _NC + lax.axis_index("c")
        base = wid * rows_per_w
        sg = (sg0, sg1)
        sw = (sw0, sw1)

        def fire(step, b):
            row0 = base + step * R
            pltpu.sync_copy(x_hbm.at[pl.ds(row0, R)], idx_v.at[b])
            for j in range(R):
                pltpu.make_async_copy(
                    p_hbm.at[idx_v.at[b, j]], rows_v.at[b, j], sg[b]
                ).start()

        def drain_gather(b):
            for j in range(R):
                pltpu.make_async_copy(
                    p_hbm.at[idx_v.at[b, j]], rows_v.at[b, j], sg[b]
                ).wait()

        def wb_start(step, b):
            row0 = base + step * R
            pltpu.make_async_copy(
                rows_v.at[b], out_hbm.at[pl.ds(row0, R)], sw[b]
            ).start()

        def wb_wait(step, b):
            row0 = base + step * R
            pltpu.make_async_copy(
                rows_v.at[b], out_hbm.at[pl.ds(row0, R)], sw[b]
            ).wait()

        fire(0, 0)
        fire(1, 1)

        def body(i, carry):
            g = 2 * i
            drain_gather(0)
            wb_start(g, 0)
            wb_wait(g, 0)
            @pl.when(i < half - 1)
            def _():
                fire(g + 2, 0)
            drain_gather(1)
            wb_start(g + 1, 1)
            wb_wait(g + 1, 1)
            @pl.when(i < half - 1)
            def _():
                fire(g + 3, 1)
            return carry

        lax.fori_loop(0, half, body, 0)

    return gather_k(x3, p)


def kernel(x, table, W, b):
    bsz, seq = x.shape
    n = bsz * seq
    emb, hid = W.shape
    vocab = table.shape[0]
    vb = 4096
    vp = ((vocab + vb - 1) // vb) * vb
    p = _tc_build_p(table.T, W.T, b.reshape(1, emb), vp, vb)
    x3 = x.astype(jnp.int32).reshape(n // _RPD, _RPD)
    out = _sc_gather(x3, p)
    return out.reshape(bsz, seq, emb)


# speedup vs baseline: 37.0976x; 1.1312x over previous
import functools

import jax
import jax.numpy as jnp
from jax import lax
from jax.experimental import pallas as pl
from jax.experimental.pallas import tpu as pltpu
from jax.experimental.pallas import tpu_sc as plsc

_NC = 2
_NS = 16
_NW = _NC * _NS
_RPD = 128


def _tc_build_p(tableT, wt, b2, vp, vb):
    hid, _ = tableT.shape
    emb = wt.shape[1]
    grid = vp // vb

    def pk(t_ref, w_ref, b_ref, o_ref):
        o_ref[...] = (
            lax.dot_general(t_ref[...], w_ref[...],
                            (((0,), (0,)), ((), ())),
                            preferred_element_type=jnp.float32)
            + b_ref[...]
        )

    return pl.pallas_call(
        pk,
        grid=(grid,),
        in_specs=[
            pl.BlockSpec((hid, vb), lambda i: (0, i)),
            pl.BlockSpec((hid, emb), lambda i: (0, 0)),
            pl.BlockSpec((1, emb), lambda i: (0, 0)),
        ],
        out_specs=pl.BlockSpec((vb, emb), lambda i: (i, 0)),
        out_shape=jax.ShapeDtypeStruct((vp, emb), jnp.float32),
    )(tableT, wt, b2)


def _sc_gather(x3, p):
    n_rows = x3.shape[0]
    emb = p.shape[1]
    rows_per_w = n_rows // _NW
    R = 2
    steps = rows_per_w // R
    half = steps // 2

    mesh = plsc.VectorSubcoreMesh(
        core_axis_name="c", subcore_axis_name="s",
        num_cores=_NC, num_subcores=_NS)

    @functools.partial(
        pl.kernel,
        out_type=jax.ShapeDtypeStruct((n_rows, _RPD, emb), jnp.float32),
        mesh=mesh,
        scratch_types=[
            pltpu.VMEM((2, R, _RPD), jnp.int32),
            pltpu.VMEM((2, R, _RPD, emb), jnp.float32),
            pltpu.SemaphoreType.DMA,
            pltpu.SemaphoreType.DMA,
            pltpu.SemaphoreType.DMA,
            pltpu.SemaphoreType.DMA,
        ],
    )
    def gather_k(x_hbm, p_hbm, out_hbm, idx_v, rows_v, sg0, sg1, sw0, sw1):
        wid = lax.axis_index("s") * _NC + lax.axis_index("c")
        base = wid * rows_per_w
        sg = (sg0, sg1)
        sw = (sw0, sw1)

        def fire(step, b):
            row0 = base + step * R
            pltpu.sync_copy(x_hbm.at[pl.ds(row0, R)], idx_v.at[b])
            for j in range(R):
                pltpu.make_async_copy(
                    p_hbm.at[idx_v.at[b, j]], rows_v.at[b, j], sg[b]
                ).start()

        def drain_gather(b):
            for j in range(R):
                pltpu.make_async_copy(
                    p_hbm.at[idx_v.at[b, j]], rows_v.at[b, j], sg[b]
                ).wait()

        def wb_start(step, b):
            row0 = base + step * R
            pltpu.make_async_copy(
                rows_v.at[b], out_hbm.at[pl.ds(row0, R)], sw[b]
            ).start()

        def wb_wait(step, b):
            row0 = base + step * R
            pltpu.make_async_copy(
                rows_v.at[b], out_hbm.at[pl.ds(row0, R)], sw[b]
            ).wait()

        fire(0, 0)
        fire(1, 1)

        def body(i, carry):
            g = 2 * i
            drain_gather(0)
            wb_start(g, 0)
            wb_wait(g, 0)
            @pl.when(i < half - 1)
            def _():
                fire(g + 2, 0)
            drain_gather(1)
            wb_start(g + 1, 1)
            wb_wait(g + 1, 1)
            @pl.when(i < half - 1)
            def _():
                fire(g + 3, 1)
            return carry

        lax.fori_loop(0, half, body, 0)

    return gather_k(x3, p)


def kernel(x, table, W, b):
    bsz, seq = x.shape
    n = bsz * seq
    emb, hid = W.shape
    vocab = table.shape[0]
    vb = 8192
    vp = ((vocab + vb - 1) // vb) * vb
    p = _tc_build_p(table.T, W.T, b.reshape(1, emb), vp, vb)
    x3 = x.astype(jnp.int32).reshape(n // _RPD, _RPD)
    out = _sc_gather(x3, p)
    return out.reshape(bsz, seq, emb)


# speedup vs baseline: 39.4821x; 1.0643x over previous
import functools

import jax
import jax.numpy as jnp
from jax import lax
from jax.experimental import pallas as pl
from jax.experimental.pallas import tpu as pltpu
from jax.experimental.pallas import tpu_sc as plsc

_NC = 2
_NS = 16
_NW = _NC * _NS
_RPD = 128


def _tc_build_p(tableT, wt, b2, vp, vb):
    hid, _ = tableT.shape
    emb = wt.shape[1]
    grid = vp // vb

    def pk(t_ref, w_ref, b_ref, o_ref):
        o_ref[...] = (
            lax.dot_general(t_ref[...], w_ref[...],
                            (((0,), (0,)), ((), ())),
                            preferred_element_type=jnp.float32)
            + b_ref[...]
        )

    return pl.pallas_call(
        pk,
        grid=(grid,),
        in_specs=[
            pl.BlockSpec((hid, vb), lambda i: (0, i)),
            pl.BlockSpec((hid, emb), lambda i: (0, 0)),
            pl.BlockSpec((1, emb), lambda i: (0, 0)),
        ],
        out_specs=pl.BlockSpec((vb, emb), lambda i: (i, 0)),
        out_shape=jax.ShapeDtypeStruct((vp, emb), jnp.float32),
    )(tableT, wt, b2)


def _sc_gather(x3, p):
    n_rows = x3.shape[0]
    emb = p.shape[1]
    rows_per_w = n_rows // _NW
    R = 2
    steps = rows_per_w // R
    half = steps // 2

    mesh = plsc.VectorSubcoreMesh(
        core_axis_name="c", subcore_axis_name="s",
        num_cores=_NC, num_subcores=_NS)

    @functools.partial(
        pl.kernel,
        out_type=jax.ShapeDtypeStruct((n_rows, _RPD, emb), jnp.float32),
        mesh=mesh,
        scratch_types=[
            pltpu.VMEM((2, R, _RPD), jnp.int32),
            pltpu.VMEM((2, R, _RPD, emb), jnp.float32),
            pltpu.SemaphoreType.DMA,
            pltpu.SemaphoreType.DMA,
            pltpu.SemaphoreType.DMA,
            pltpu.SemaphoreType.DMA,
        ],
    )
    def gather_k(x_hbm, p_hbm, out_hbm, idx_v, rows_v, sg0, sg1, sw0, sw1):
        wid = lax.axis_index("s") * _NC + lax.axis_index("c")
        base = wid * rows_per_w
        sg = (sg0, sg1)
        sw = (sw0, sw1)

        def fire(step, b):
            row0 = base + step * R
            pltpu.sync_copy(x_hbm.at[pl.ds(row0, R)], idx_v.at[b])
            for j in range(R):
                pltpu.make_async_copy(
                    p_hbm.at[idx_v.at[b, j]], rows_v.at[b, j], sg[b]
                ).start()

        def drain_gather(b):
            for j in range(R):
                pltpu.make_async_copy(
                    p_hbm.at[idx_v.at[b, j]], rows_v.at[b, j], sg[b]
                ).wait()

        def wb_start(step, b):
            row0 = base + step * R
            pltpu.make_async_copy(
                rows_v.at[b], out_hbm.at[pl.ds(row0, R)], sw[b]
            ).start()

        def wb_wait(step, b):
            row0 = base + step * R
            pltpu.make_async_copy(
                rows_v.at[b], out_hbm.at[pl.ds(row0, R)], sw[b]
            ).wait()

        fire(0, 0)
        fire(1, 1)

        def body(i, carry):
            g = 2 * i
            drain_gather(0)
            wb_start(g, 0)
            wb_wait(g, 0)
            @pl.when(i < half - 1)
            def _():
                fire(g + 2, 0)
            drain_gather(1)
            wb_start(g + 1, 1)
            wb_wait(g + 1, 1)
            @pl.when(i < half - 1)
            def _():
                fire(g + 3, 1)
            return carry

        lax.fori_loop(0, half, body, 0)

    return gather_k(x3, p)


def kernel(x, table, W, b):
    bsz, seq = x.shape
    n = bsz * seq
    emb, hid = W.shape
    vocab = table.shape[0]
    vb = 16384
    vp = ((vocab + vb - 1) // vb) * vb
    p = _tc_build_p(table.T, W.T, b.reshape(1, emb), vp, vb)
    x3 = x.astype(jnp.int32).reshape(n // _RPD, _RPD)
    out = _sc_gather(x3, p)
    return out.reshape(bsz, seq, emb)


# speedup vs baseline: 39.9021x; 1.0106x over previous
import functools

import jax
import jax.numpy as jnp
from jax import lax
from jax.experimental import pallas as pl
from jax.experimental.pallas import tpu as pltpu
from jax.experimental.pallas import tpu_sc as plsc

_NC = 2
_NS = 16
_NW = _NC * _NS
_RPD = 128


def _tc_build_p(tableT, wt, b2, vp, vb):
    hid, _ = tableT.shape
    emb = wt.shape[1]
    grid = vp // vb

    def pk(t_ref, w_ref, b_ref, o_ref):
        o_ref[...] = (
            lax.dot_general(t_ref[...], w_ref[...],
                            (((0,), (0,)), ((), ())),
                            preferred_element_type=jnp.float32)
            + b_ref[...]
        )

    return pl.pallas_call(
        pk,
        grid=(grid,),
        in_specs=[
            pl.BlockSpec((hid, vb), lambda i: (0, i)),
            pl.BlockSpec((hid, emb), lambda i: (0, 0)),
            pl.BlockSpec((1, emb), lambda i: (0, 0)),
        ],
        out_specs=pl.BlockSpec((vb, emb), lambda i: (i, 0)),
        out_shape=jax.ShapeDtypeStruct((vp, emb), jnp.float32),
    )(tableT, wt, b2)


def _sc_gather(x3, p):
    n_rows = x3.shape[0]
    emb = p.shape[1]
    rows_per_w = n_rows // _NW
    R = 2
    steps = rows_per_w // R
    half = steps // 2

    mesh = plsc.VectorSubcoreMesh(
        core_axis_name="c", subcore_axis_name="s",
        num_cores=_NC, num_subcores=_NS)

    @functools.partial(
        pl.kernel,
        out_type=jax.ShapeDtypeStruct((n_rows, _RPD, emb), jnp.float32),
        mesh=mesh,
        scratch_types=[
            pltpu.VMEM((2, R, _RPD), jnp.int32),
            pltpu.VMEM((2, R, _RPD, emb), jnp.float32),
            pltpu.SemaphoreType.DMA,
            pltpu.SemaphoreType.DMA,
            pltpu.SemaphoreType.DMA,
            pltpu.SemaphoreType.DMA,
        ],
    )
    def gather_k(x_hbm, p_hbm, out_hbm, idx_v, rows_v, sg0, sg1, sw0, sw1):
        wid = lax.axis_index("s") * _NC + lax.axis_index("c")
        base = wid * rows_per_w
        sg = (sg0, sg1)
        sw = (sw0, sw1)

        def fire(step, b):
            row0 = base + step * R
            pltpu.sync_copy(x_hbm.at[pl.ds(row0, R)], idx_v.at[b])
            for j in range(R):
                pltpu.make_async_copy(
                    p_hbm.at[idx_v.at[b, j]], rows_v.at[b, j], sg[b]
                ).start()

        def drain_gather(b):
            for j in range(R):
                pltpu.make_async_copy(
                    p_hbm.at[idx_v.at[b, j]], rows_v.at[b, j], sg[b]
                ).wait()

        def wb_start(step, b):
            row0 = base + step * R
            pltpu.make_async_copy(
                rows_v.at[b], out_hbm.at[pl.ds(row0, R)], sw[b]
            ).start()

        def wb_wait(step, b):
            row0 = base + step * R
            pltpu.make_async_copy(
                rows_v.at[b], out_hbm.at[pl.ds(row0, R)], sw[b]
            ).wait()

        fire(0, 0)
        fire(1, 1)

        def body(i, carry):
            g = 2 * i
            drain_gather(0)
            wb_start(g, 0)
            wb_wait(g, 0)
            @pl.when(i < half - 1)
            def _():
                fire(g + 2, 0)
            drain_gather(1)
            wb_start(g + 1, 1)
            wb_wait(g + 1, 1)
            @pl.when(i < half - 1)
            def _():
                fire(g + 3, 1)
            return carry

        lax.fori_loop(0, half, body, 0)

    return gather_k(x3, p)


def kernel(x, table, W, b):
    bsz, seq = x.shape
    n = bsz * seq
    emb, hid = W.shape
    vocab = table.shape[0]
    vb = 32768
    vp = ((vocab + vb - 1) // vb) * vb
    p = _tc_build_p(table.T, W.T, b.reshape(1, emb), vp, vb)
    x3 = x.astype(jnp.int32).reshape(n // _RPD, _RPD)
    out = _sc_gather(x3, p)
    return out.reshape(bsz, seq, emb)
